# baseline (speedup 1.0000x reference)
import jax
import jax.numpy as jnp
from jax.experimental import pallas as pl
from jax.experimental.pallas import tpu as pltpu

_CH = 2000
_NBUF = 6


def _scale_body(ei_ref, aw_ref, x_hbm, o_hbm, ibuf, obuf, w_ref, isem, osem):
    m, _, c = x_hbm.shape
    n = x_hbm.shape[1]
    num_l = aw_ref.shape[0]
    nsplit = n // _CH
    total = m * nsplit

    ei = ei_ref[...]
    onehot = (ei == jax.lax.broadcasted_iota(jnp.int32, (m, num_l), 1))
    w_ref[...] = jax.lax.dot_general(
        onehot.astype(jnp.float32), aw_ref[...],
        (((1,), (0,)), ((), ())),
        preferred_element_type=jnp.float32)

    def in_copy(t, b):
        return pltpu.make_async_copy(
            x_hbm.at[t // nsplit, pl.ds((t % nsplit) * _CH, _CH)],
            ibuf.at[b], isem.at[b])

    def out_copy(t, b):
        return pltpu.make_async_copy(
            obuf.at[b],
            o_hbm.at[t // nsplit, pl.ds((t % nsplit) * _CH, _CH)],
            osem.at[b])

    for b in range(min(_NBUF, total)):
        in_copy(b, b).start()

    for t in range(total):
        b = t % _NBUF
        in_copy(t, b).wait()
        if t >= _NBUF:
            out_copy(t - _NBUF, b).wait()
        obuf[b] = ibuf[b] * w_ref[pl.ds(t // nsplit, 1), :]
        if t + _NBUF < total:
            in_copy(t + _NBUF, b).start()
        out_copy(t, b).start()

    for t in range(max(total - _NBUF, 0), total):
        out_copy(t, t % _NBUF).wait()


def kernel(node_input, affine_weight, expand_index):
    n, m, c = node_input.shape
    x_t = jnp.transpose(node_input, (1, 0, 2))
    aw = affine_weight.reshape(affine_weight.shape[-2], c)
    ei = expand_index.astype(jnp.int32).reshape(m, 1)

    out_t = pl.pallas_call(
        _scale_body,
        in_specs=[
            pl.BlockSpec(memory_space=pltpu.MemorySpace.VMEM),
            pl.BlockSpec(memory_space=pltpu.MemorySpace.VMEM),
            pl.BlockSpec(memory_space=pltpu.MemorySpace.HBM),
        ],
        out_specs=pl.BlockSpec(memory_space=pltpu.MemorySpace.HBM),
        out_shape=jax.ShapeDtypeStruct((m, n, c), jnp.float32),
        scratch_shapes=[
            pltpu.VMEM((_NBUF, _CH, c), jnp.float32),
            pltpu.VMEM((_NBUF, _CH, c), jnp.float32),
            pltpu.VMEM((m, c), jnp.float32),
            pltpu.SemaphoreType.DMA((_NBUF,)),
            pltpu.SemaphoreType.DMA((_NBUF,)),
        ],
    )(ei, aw, x_t)
    return jnp.transpose(out_t, (1, 0, 2))


# speedup vs baseline: 1.0212x; 1.0212x over previous
import jax
import jax.numpy as jnp
from jax.experimental import pallas as pl
from jax.experimental.pallas import tpu as pltpu

_MBLK = 2


def _scale_body(ei_ref, aw_ref, x_ref, o_ref, w_ref):
    m = ei_ref.shape[0]

    @pl.when(pl.program_id(0) == 0)
    def _():
        for mm in range(m):
            l = ei_ref[mm]
            w_ref[pl.ds(mm, 1), :] = aw_ref[pl.ds(l, 1), :]

    i = pl.program_id(0)
    o_ref[...] = x_ref[...] * w_ref[pl.ds(i * _MBLK, _MBLK), :][:, None, :]


def kernel(node_input, affine_weight, expand_index):
    n, m, c = node_input.shape
    x_t = jnp.transpose(node_input, (1, 0, 2))
    aw = affine_weight.reshape(affine_weight.shape[-2], c)
    ei = expand_index.astype(jnp.int32)

    mb = _MBLK
    out_t = pl.pallas_call(
        _scale_body,
        grid=((m + mb - 1) // mb,),
        in_specs=[
            pl.BlockSpec(memory_space=pltpu.MemorySpace.SMEM),
            pl.BlockSpec(aw.shape, lambda i: (0, 0)),
            pl.BlockSpec((mb, n, c), lambda i: (i, 0, 0)),
        ],
        out_specs=pl.BlockSpec((mb, n, c), lambda i: (i, 0, 0)),
        out_shape=jax.ShapeDtypeStruct((m, n, c), jnp.float32),
        scratch_shapes=[pltpu.VMEM((m + m % mb, c), jnp.float32)],
    )(ei, aw, x_t)
    return jnp.transpose(out_t, (1, 0, 2))
